# initial kernel scaffold (unmeasured)
import jax
import jax.numpy as jnp
from jax import lax
from jax.experimental import pallas as pl
from jax.experimental.pallas import tpu as pltpu

N_DEV = 16
SQ = 2048
D_MODEL = 1024
H_PER = 8
DH = 128
H_SLICE = H_PER * DH
CHUNK = SQ // N_DEV
BLK = 64
SCALE = 0.08838834764831843

N_STEP = N_DEV - 1


def _wrap(v):
    return lax.rem(v + 2 * N_DEV, N_DEV)


def _allreduce_body(x_ref, out_ref, rs_bufs, ag_bufs,
                    rs_send_sems, rs_recv_sems, ag_send_sems, ag_recv_sems):
    pos = lax.axis_index("i")
    right = _wrap(pos + 1)

    out_ref[...] = x_ref[...]

    for s in range(N_STEP):
        c_send = _wrap(pos - s)
        rdma = pltpu.make_async_remote_copy(
            src_ref=out_ref.at[pl.ds(c_send * CHUNK, CHUNK), :],
            dst_ref=rs_bufs.at[s],
            send_sem=rs_send_sems.at[s],
            recv_sem=rs_recv_sems.at[s],
            device_id=(right,),
            device_id_type=pl.DeviceIdType.MESH,
        )
        rdma.start()
        rdma.wait()
        c_recv = _wrap(pos - s - 1)
        idx = pl.ds(c_recv * CHUNK, CHUNK)
        out_ref[idx, :] = out_ref[idx, :] + rs_bufs[s]

    for s in range(N_STEP):
        if s == 0:
            c_send = _wrap(pos + 1)
            src = out_ref.at[pl.ds(c_send * CHUNK, CHUNK), :]
        else:
            src = ag_bufs.at[s - 1]
        rdma = pltpu.make_async_remote_copy(
            src_ref=src,
            dst_ref=ag_bufs.at[s],
            send_sem=ag_send_sems.at[s],
            recv_sem=ag_recv_sems.at[s],
            device_id=(right,),
            device_id_type=pl.DeviceIdType.MESH,
        )
        rdma.start()
        rdma.wait()
        c_recv = _wrap(pos - s)
        out_ref[pl.ds(c_recv * CHUNK, CHUNK), :] = ag_bufs[s]


def _ring_allreduce(partial):
    return pl.pallas_call(
        _allreduce_body,
        out_shape=jax.ShapeDtypeStruct((SQ, D_MODEL), jnp.float32),
        in_specs=[pl.BlockSpec(memory_space=pltpu.VMEM)],
        out_specs=pl.BlockSpec(memory_space=pltpu.VMEM),
        scratch_shapes=[
            pltpu.VMEM((N_STEP, CHUNK, D_MODEL), jnp.float32),
            pltpu.VMEM((N_STEP, CHUNK, D_MODEL), jnp.float32),
            pltpu.SemaphoreType.DMA((N_STEP,)),
            pltpu.SemaphoreType.DMA((N_STEP,)),
            pltpu.SemaphoreType.DMA((N_STEP,)),
            pltpu.SemaphoreType.DMA((N_STEP,)),
        ],
        compiler_params=pltpu.CompilerParams(collective_id=0),
    )(partial)


def kernel(x, Wq, K_ext, V_ext, Wo):
    pos = lax.axis_index("i")
    bf = jnp.bfloat16

    x2 = x[0]
    Wq_l = lax.dynamic_slice_in_dim(Wq, pos * H_SLICE, H_SLICE, axis=1)
    q = jnp.dot(x2.astype(bf), Wq_l.astype(bf),
                preferred_element_type=jnp.float32)
    q = q.reshape(SQ, H_PER, DH).astype(bf)

    k = K_ext[0].astype(bf)
    v = V_ext[0].astype(bf)

    scores = jnp.einsum("ihd,jhd->hij", q, k,
                        preferred_element_type=jnp.float32) * SCALE
    qb = jnp.arange(SQ) // BLK
    mask = qb[:, None] >= qb[None, :]
    scores = jnp.where(mask[None, :, :], scores, -1e9)
    m = scores.max(axis=-1, keepdims=True)
    w = jnp.exp(scores - m)
    w = w / w.sum(axis=-1, keepdims=True)

    ctx = jnp.einsum("hij,jhd->ihd", w.astype(bf), v,
                     preferred_element_type=jnp.float32)
    ctx = ctx.reshape(SQ, H_SLICE)

    Wo_l = lax.dynamic_slice_in_dim(Wo, pos * H_SLICE, H_SLICE, axis=0)
    partial = jnp.dot(ctx.astype(bf), Wo_l.astype(bf),
                      preferred_element_type=jnp.float32)

    out = _ring_allreduce(partial)
    return out[None, :, :]


# baseline (device time: 370137 ns/iter reference)
import jax
import jax.numpy as jnp
from jax import lax
from jax.experimental import pallas as pl
from jax.experimental.pallas import tpu as pltpu

N_DEV = 16
SQ = 2048
D_MODEL = 1024
H_PER = 8
DH = 128
H_SLICE = H_PER * DH
CHUNK = SQ // N_DEV
BLK = 64
SCALE = 0.08838834764831843

N_STEP = N_DEV - 1


def _wrap(v):
    return lax.rem(v + 2 * N_DEV, N_DEV)


def _allreduce_body(x_ref, out_ref, rs_bufs, ag_bufs,
                    rs_send_sems, rs_recv_sems, ag_send_sems, ag_recv_sems):
    pos = lax.axis_index("i")
    right = _wrap(pos + 1)

    out_ref[...] = x_ref[...]

    for s in range(N_STEP):
        c_send = _wrap(pos - s)
        rdma = pltpu.make_async_remote_copy(
            src_ref=out_ref.at[pl.ds(c_send * CHUNK, CHUNK), :],
            dst_ref=rs_bufs.at[s],
            send_sem=rs_send_sems.at[s],
            recv_sem=rs_recv_sems.at[s],
            device_id=(right,),
            device_id_type=pl.DeviceIdType.MESH,
        )
        rdma.start()
        rdma.wait()
        c_recv = _wrap(pos - s - 1)
        idx = pl.ds(c_recv * CHUNK, CHUNK)
        out_ref[idx, :] = out_ref[idx, :] + rs_bufs[s]

    for s in range(N_STEP):
        if s == 0:
            c_send = _wrap(pos + 1)
            src = out_ref.at[pl.ds(c_send * CHUNK, CHUNK), :]
        else:
            src = ag_bufs.at[s - 1]
        rdma = pltpu.make_async_remote_copy(
            src_ref=src,
            dst_ref=ag_bufs.at[s],
            send_sem=ag_send_sems.at[s],
            recv_sem=ag_recv_sems.at[s],
            device_id=(right,),
            device_id_type=pl.DeviceIdType.MESH,
        )
        rdma.start()
        rdma.wait()
        c_recv = _wrap(pos - s)
        out_ref[pl.ds(c_recv * CHUNK, CHUNK), :] = ag_bufs[s]


def _ring_allreduce(partial):
    return pl.pallas_call(
        _allreduce_body,
        out_shape=jax.ShapeDtypeStruct((SQ, D_MODEL), jnp.float32),
        in_specs=[pl.BlockSpec(memory_space=pltpu.VMEM)],
        out_specs=pl.BlockSpec(memory_space=pltpu.VMEM),
        scratch_shapes=[
            pltpu.VMEM((N_STEP, CHUNK, D_MODEL), jnp.float32),
            pltpu.VMEM((N_STEP, CHUNK, D_MODEL), jnp.float32),
            pltpu.SemaphoreType.DMA((N_STEP,)),
            pltpu.SemaphoreType.DMA((N_STEP,)),
            pltpu.SemaphoreType.DMA((N_STEP,)),
            pltpu.SemaphoreType.DMA((N_STEP,)),
        ],
    )(partial)


def kernel(x, Wq, K_ext, V_ext, Wo):
    pos = lax.axis_index("i")
    bf = jnp.bfloat16

    x2 = x[0]
    Wq_l = lax.dynamic_slice_in_dim(Wq, pos * H_SLICE, H_SLICE, axis=1)
    q = jnp.dot(x2.astype(bf), Wq_l.astype(bf),
                preferred_element_type=jnp.float32)
    q = q.reshape(SQ, H_PER, DH).astype(bf)

    k = K_ext[0].astype(bf)
    v = V_ext[0].astype(bf)

    scores = jnp.einsum("ihd,jhd->hij", q, k,
                        preferred_element_type=jnp.float32) * SCALE
    qb = jnp.arange(SQ) // BLK
    mask = qb[:, None] >= qb[None, :]
    scores = jnp.where(mask[None, :, :], scores, -1e9)
    m = scores.max(axis=-1, keepdims=True)
    w = jnp.exp(scores - m)
    w = w / w.sum(axis=-1, keepdims=True)

    ctx = jnp.einsum("hij,jhd->ihd", w.astype(bf), v,
                     preferred_element_type=jnp.float32)
    ctx = ctx.reshape(SQ, H_SLICE)

    Wo_l = lax.dynamic_slice_in_dim(Wo, pos * H_SLICE, H_SLICE, axis=0)
    partial = jnp.dot(ctx.astype(bf), Wo_l.astype(bf),
                      preferred_element_type=jnp.float32)

    out = _ring_allreduce(partial)
    return out[None, :, :]


# device time: 357530 ns/iter; 1.0353x vs baseline; 1.0353x over previous
import jax
import jax.numpy as jnp
from jax import lax
from jax.experimental import pallas as pl
from jax.experimental.pallas import tpu as pltpu

N_DEV = 16
SQ = 2048
D_MODEL = 1024
H_PER = 8
DH = 128
H_SLICE = H_PER * DH
CHUNK = SQ // N_DEV
BLK = 64
SCALE = 0.08838834764831843

N_STEP = N_DEV - 1


def _wrap(v):
    return lax.rem(v + 2 * N_DEV, N_DEV)


def _allreduce_body(x_ref, out_ref, rs_bufs, ag_bufs,
                    rs_send_sems, rs_recv_sems, ag_send_sems, ag_recv_sems):
    pos = lax.axis_index("i")
    right = _wrap(pos + 1)

    out_ref[...] = x_ref[...]

    for s in range(N_STEP):
        c_send = _wrap(pos - s)
        rdma = pltpu.make_async_remote_copy(
            src_ref=out_ref.at[pl.ds(c_send * CHUNK, CHUNK), :],
            dst_ref=rs_bufs.at[s],
            send_sem=rs_send_sems.at[s],
            recv_sem=rs_recv_sems.at[s],
            device_id=(right,),
            device_id_type=pl.DeviceIdType.MESH,
        )
        rdma.start()
        rdma.wait()
        c_recv = _wrap(pos - s - 1)
        idx = pl.ds(c_recv * CHUNK, CHUNK)
        out_ref[idx, :] = out_ref[idx, :] + rs_bufs[s]

    for s in range(N_STEP):
        if s == 0:
            c_send = _wrap(pos + 1)
            src = out_ref.at[pl.ds(c_send * CHUNK, CHUNK), :]
        else:
            src = ag_bufs.at[s - 1]
        rdma = pltpu.make_async_remote_copy(
            src_ref=src,
            dst_ref=ag_bufs.at[s],
            send_sem=ag_send_sems.at[s],
            recv_sem=ag_recv_sems.at[s],
            device_id=(right,),
            device_id_type=pl.DeviceIdType.MESH,
        )
        rdma.start()
        rdma.wait()
        c_recv = _wrap(pos - s)
        out_ref[pl.ds(c_recv * CHUNK, CHUNK), :] = ag_bufs[s]


def _ring_allreduce(partial):
    return pl.pallas_call(
        _allreduce_body,
        out_shape=jax.ShapeDtypeStruct((SQ, D_MODEL), jnp.float32),
        in_specs=[pl.BlockSpec(memory_space=pltpu.VMEM)],
        out_specs=pl.BlockSpec(memory_space=pltpu.VMEM),
        scratch_shapes=[
            pltpu.VMEM((N_STEP, CHUNK, D_MODEL), jnp.float32),
            pltpu.VMEM((N_STEP, CHUNK, D_MODEL), jnp.float32),
            pltpu.SemaphoreType.DMA((N_STEP,)),
            pltpu.SemaphoreType.DMA((N_STEP,)),
            pltpu.SemaphoreType.DMA((N_STEP,)),
            pltpu.SemaphoreType.DMA((N_STEP,)),
        ],
    )(partial)


QT = 256
NQ = SQ // QT


def _attn_body(q_ref, k_ref, v_ref, o_ref):
    i = pl.program_id(1)
    q = q_ref[...]
    s = lax.dot_general(q, k_ref[...], (((1,), (1,)), ((), ())),
                        preferred_element_type=jnp.float32) * SCALE
    row = i * QT + lax.broadcasted_iota(jnp.int32, (QT, SQ), 0)
    col = lax.broadcasted_iota(jnp.int32, (QT, SQ), 1)
    keep = (col // BLK) <= (row // BLK)
    s = jnp.where(keep, s, -1e9)
    m = jnp.max(s, axis=1, keepdims=True)
    p = jnp.exp(s - m)
    p = p / jnp.sum(p, axis=1, keepdims=True)
    o_ref[...] = lax.dot_general(p.astype(jnp.bfloat16), v_ref[...],
                                 (((1,), (0,)), ((), ())),
                                 preferred_element_type=jnp.float32
                                 ).astype(o_ref.dtype)


def _attention(q, k, v):
    return pl.pallas_call(
        _attn_body,
        out_shape=jax.ShapeDtypeStruct((SQ, H_SLICE), jnp.bfloat16),
        grid=(H_PER, NQ),
        in_specs=[
            pl.BlockSpec((QT, DH), lambda h, i: (i, h)),
            pl.BlockSpec((SQ, DH), lambda h, i: (0, h)),
            pl.BlockSpec((SQ, DH), lambda h, i: (0, h)),
        ],
        out_specs=pl.BlockSpec((QT, DH), lambda h, i: (i, h)),
    )(q, k, v)


def kernel(x, Wq, K_ext, V_ext, Wo):
    pos = lax.axis_index("i")
    bf = jnp.bfloat16

    x2 = x[0]
    Wq_l = lax.dynamic_slice_in_dim(Wq, pos * H_SLICE, H_SLICE, axis=1)
    q = jnp.dot(x2.astype(bf), Wq_l.astype(bf),
                preferred_element_type=jnp.float32)

    k = K_ext[0].reshape(SQ, H_SLICE).astype(bf)
    v = V_ext[0].reshape(SQ, H_SLICE).astype(bf)

    ctx = _attention(q.astype(bf), k, v)

    Wo_l = lax.dynamic_slice_in_dim(Wo, pos * H_SLICE, H_SLICE, axis=0)
    partial = jnp.dot(ctx, Wo_l.astype(bf),
                      preferred_element_type=jnp.float32)

    out = _ring_allreduce(partial)
    return out[None, :, :]


# device time: 235498 ns/iter; 1.5717x vs baseline; 1.5182x over previous
import jax
import jax.numpy as jnp
from jax import lax
from jax.experimental import pallas as pl
from jax.experimental.pallas import tpu as pltpu

N_DEV = 16
SQ = 2048
D_MODEL = 1024
H_PER = 8
DH = 128
H_SLICE = H_PER * DH
CHUNK = SQ // N_DEV
BLK = 64
SCALE = 0.08838834764831843

N_STEP = N_DEV - 1


def _allreduce_body(x_ref, out_ref, g_ref, rb0, rb1, rb2, rb3,
                    rs_send_sems, rs_recv_sems, ag_send_sems, ag_recv_sems):
    pos = lax.axis_index("i")
    w = lax.rem(pos, 4)
    z = pos // 4
    b_x = jnp.where((w == 1) | (w == 2), 1, 0).astype(jnp.int32)
    b_y = w // 2
    b_z0 = lax.rem(z, 2)
    b_z1 = z // 2
    p_x = pos + 1 - 2 * lax.rem(w, 2)
    p_y = pos + 3 - 2 * w
    p_z0 = pos + (1 - 2 * b_z0) * 4
    p_z1 = pos + (1 - 2 * b_z1) * 8

    out_ref[...] = x_ref[...]

    active = jnp.int32(0)
    rs_steps = [(8, b_x, p_x, rb0), (4, b_y, p_y, rb1),
                (2, b_z0, p_z0, rb2), (1, b_z1, p_z1, rb3)]
    for k, (h, b, partner, rbuf) in enumerate(rs_steps):
        keep = active + b * h
        send = active + (1 - b) * h
        s_sl = pl.ds(send * CHUNK, h * CHUNK)
        g_ref[s_sl, :] = out_ref[s_sl, :].astype(jnp.bfloat16)
        rdma = pltpu.make_async_remote_copy(
            src_ref=g_ref.at[s_sl, :],
            dst_ref=rbuf,
            send_sem=rs_send_sems.at[k],
            recv_sem=rs_recv_sems.at[k],
            device_id=(partner,),
            device_id_type=pl.DeviceIdType.MESH,
        )
        rdma.start()
        rdma.wait()
        k_sl = pl.ds(keep * CHUNK, h * CHUNK)
        out_ref[k_sl, :] = out_ref[k_sl, :] + rbuf[...].astype(jnp.float32)
        active = keep

    own_sl = pl.ds(active * CHUNK, CHUNK)
    g_ref[own_sl, :] = out_ref[own_sl, :].astype(jnp.bfloat16)

    ag_steps = [(1, b_z1, p_z1), (2, b_z0, p_z0),
                (4, b_y, p_y), (8, b_x, p_x)]
    for k, (sz, b, partner) in enumerate(ag_steps):
        new = active - b * sz
        other = new + (1 - b) * sz
        my_sl = pl.ds(active * CHUNK, sz * CHUNK)
        rdma = pltpu.make_async_remote_copy(
            src_ref=g_ref.at[my_sl, :],
            dst_ref=g_ref.at[my_sl, :],
            send_sem=ag_send_sems.at[k],
            recv_sem=ag_recv_sems.at[k],
            device_id=(partner,),
            device_id_type=pl.DeviceIdType.MESH,
        )
        rdma.start()
        rdma.wait()
        o_sl = pl.ds(other * CHUNK, sz * CHUNK)
        out_ref[o_sl, :] = g_ref[o_sl, :].astype(jnp.float32)
        active = new


def _ring_allreduce(partial):
    return pl.pallas_call(
        _allreduce_body,
        out_shape=jax.ShapeDtypeStruct((SQ, D_MODEL), jnp.float32),
        in_specs=[pl.BlockSpec(memory_space=pltpu.VMEM)],
        out_specs=pl.BlockSpec(memory_space=pltpu.VMEM),
        scratch_shapes=[
            pltpu.VMEM((SQ, D_MODEL), jnp.bfloat16),
            pltpu.VMEM((8 * CHUNK, D_MODEL), jnp.bfloat16),
            pltpu.VMEM((4 * CHUNK, D_MODEL), jnp.bfloat16),
            pltpu.VMEM((2 * CHUNK, D_MODEL), jnp.bfloat16),
            pltpu.VMEM((1 * CHUNK, D_MODEL), jnp.bfloat16),
            pltpu.SemaphoreType.DMA((4,)),
            pltpu.SemaphoreType.DMA((4,)),
            pltpu.SemaphoreType.DMA((4,)),
            pltpu.SemaphoreType.DMA((4,)),
        ],
    )(partial)


QT = 256
NQ = SQ // QT


def _attn_body(q_ref, k_ref, v_ref, o_ref):
    i = pl.program_id(1)
    q = q_ref[...]
    s = lax.dot_general(q, k_ref[...], (((1,), (1,)), ((), ())),
                        preferred_element_type=jnp.float32) * SCALE
    row = i * QT + lax.broadcasted_iota(jnp.int32, (QT, SQ), 0)
    col = lax.broadcasted_iota(jnp.int32, (QT, SQ), 1)
    keep = (col // BLK) <= (row // BLK)
    s = jnp.where(keep, s, -1e9)
    m = jnp.max(s, axis=1, keepdims=True)
    p = jnp.exp(s - m)
    p = p / jnp.sum(p, axis=1, keepdims=True)
    o_ref[...] = lax.dot_general(p.astype(jnp.bfloat16), v_ref[...],
                                 (((1,), (0,)), ((), ())),
                                 preferred_element_type=jnp.float32
                                 ).astype(o_ref.dtype)


def _attention(q, k, v):
    return pl.pallas_call(
        _attn_body,
        out_shape=jax.ShapeDtypeStruct((SQ, H_SLICE), jnp.bfloat16),
        grid=(H_PER, NQ),
        in_specs=[
            pl.BlockSpec((QT, DH), lambda h, i: (i, h)),
            pl.BlockSpec((SQ, DH), lambda h, i: (0, h)),
            pl.BlockSpec((SQ, DH), lambda h, i: (0, h)),
        ],
        out_specs=pl.BlockSpec((QT, DH), lambda h, i: (i, h)),
    )(q, k, v)


def kernel(x, Wq, K_ext, V_ext, Wo):
    pos = lax.axis_index("i")
    bf = jnp.bfloat16

    x2 = x[0]
    Wq_l = lax.dynamic_slice_in_dim(Wq, pos * H_SLICE, H_SLICE, axis=1)
    q = jnp.dot(x2.astype(bf), Wq_l.astype(bf),
                preferred_element_type=jnp.float32)

    k = K_ext[0].reshape(SQ, H_SLICE).astype(bf)
    v = V_ext[0].reshape(SQ, H_SLICE).astype(bf)

    ctx = _attention(q.astype(bf), k, v)

    Wo_l = lax.dynamic_slice_in_dim(Wo, pos * H_SLICE, H_SLICE, axis=0)
    partial = jnp.dot(ctx, Wo_l.astype(bf),
                      preferred_element_type=jnp.float32)

    out = _ring_allreduce(partial)
    return out[None, :, :]
